# baseline (speedup 1.0000x reference)
import functools

import jax
import jax.numpy as jnp
from jax import lax
from jax.experimental import pallas as pl
from jax.experimental.pallas import tpu as pltpu
from jax.experimental.pallas import tpu_sc as plsc

M = 1000000
D = 32
B = 16384

NC = 2
NS = 16
L = 16
NW = NC * NS
ROWS_MAIN = 31248
ROWS_MAX = ROWS_MAIN + 8
NSEL_GROUPS = B // L

CAP = 1024
CHUNK = 128
WCAP = CAP + CHUNK


def _body(data_hbm, idx_hbm, upd_hbm, out_hbm,
          idx_v, blist, loclist, b_w, loc_w, claim, gbuf, brow, locrow,
          scopy, sgather, sscatter):
    wid = lax.axis_index("s") * NC + lax.axis_index("c")
    lo = wid * ROWS_MAIN + 8 * (wid // 4)
    hi = (wid + 1) * ROWS_MAIN + 8 * ((wid + 1) // 4)

    cp = pltpu.async_copy(data_hbm.at[pl.ds(lo, ROWS_MAIN)],
                          out_hbm.at[pl.ds(lo, ROWS_MAIN)], scopy)

    pltpu.sync_copy(idx_hbm, idx_v)

    lane = lax.iota(jnp.int32, L)

    def sel_body(g, off):
        idxv = idx_v[pl.ds(g * L, L)]
        m = (idxv >= lo) & (idxv < hi)
        cnt = jnp.sum(m.astype(jnp.int32))
        safe = jnp.minimum(off, CAP)
        plsc.store_compressed(blist.at[pl.ds(safe, L)], g * L + lane, mask=m)
        plsc.store_compressed(loclist.at[pl.ds(safe, L)], idxv, mask=m)
        return off + cnt

    n_sel = lax.fori_loop(0, NSEL_GROUPS, sel_body, jnp.int32(0))
    n_sel = jnp.minimum(n_sel, CAP)

    def claim_body(g, _):
        jv = g * L + lane
        valid = jv < n_sel
        locv = loclist[pl.ds(g * L, L)]
        locl = jnp.where(valid, locv - lo, 0)
        for i in range(L):
            plsc.store_scatter(claim, [locl], jv, mask=valid & (lane == i))
        return 0

    lax.fori_loop(0, (n_sel + L - 1) // L, claim_body, 0)

    def win_body(g, offw):
        jv = g * L + lane
        valid = jv < n_sel
        locv = loclist[pl.ds(g * L, L)]
        bv = blist[pl.ds(g * L, L)]
        locl = jnp.where(valid, locv - lo, 0)
        cl = plsc.load_gather(claim, [locl], mask=valid)
        win = valid & (cl == jv)
        cnt = jnp.sum(win.astype(jnp.int32))
        plsc.store_compressed(b_w.at[pl.ds(offw, L)], bv, mask=win)
        plsc.store_compressed(loc_w.at[pl.ds(offw, L)], locv, mask=win)
        return offw + cnt

    n_w = lax.fori_loop(0, (n_sel + L - 1) // L, win_body, jnp.int32(0))

    nch = (n_w + CHUNK - 1) // CHUNK
    b0v = b_w[pl.ds(0, L)]
    l0v = loc_w[pl.ds(0, L)]
    b0 = jnp.full((L,), b0v[0], dtype=jnp.int32)
    l0 = jnp.full((L,), l0v[0], dtype=jnp.int32)
    for g in range(CHUNK // L):
        b_w[pl.ds(n_w + g * L, L)] = b0
        loc_w[pl.ds(n_w + g * L, L)] = l0

    @pl.when(hi - lo > ROWS_MAIN)
    def _tail():
        pltpu.sync_copy(data_hbm.at[pl.ds(lo + ROWS_MAIN, 8)],
                        out_hbm.at[pl.ds(lo + ROWS_MAIN, 8)])

    cp.wait()

    def apply_body(c, _):
        base = c * CHUNK
        for i in range(CHUNK // L):
            brow[pl.ds(i * L, L)] = b_w[pl.ds(base + i * L, L)]
            locrow[pl.ds(i * L, L)] = loc_w[pl.ds(base + i * L, L)]
        pltpu.async_copy(upd_hbm.at[brow], gbuf, sgather).wait()
        pltpu.async_copy(gbuf, out_hbm.at[locrow], sscatter).wait()
        return 0

    lax.fori_loop(0, nch, apply_body, 0)


@functools.partial(
    pl.kernel,
    out_type=jax.ShapeDtypeStruct((M, D), jnp.float32),
    mesh=plsc.VectorSubcoreMesh(
        core_axis_name="c", subcore_axis_name="s", num_cores=NC,
        num_subcores=NS),
    scratch_types=[
        pltpu.VMEM((B,), jnp.int32),
        pltpu.VMEM((CAP + L,), jnp.int32),
        pltpu.VMEM((CAP + L,), jnp.int32),
        pltpu.VMEM((WCAP,), jnp.int32),
        pltpu.VMEM((WCAP,), jnp.int32),
        pltpu.VMEM((ROWS_MAX,), jnp.int32),
        pltpu.VMEM((CHUNK, D), jnp.float32),
        pltpu.VMEM((CHUNK,), jnp.int32),
        pltpu.VMEM((CHUNK,), jnp.int32),
        pltpu.SemaphoreType.DMA,
        pltpu.SemaphoreType.DMA,
        pltpu.SemaphoreType.DMA,
    ],
    compiler_params=pltpu.CompilerParams(
        needs_layout_passes=False, use_tc_tiling_on_sc=False),
)
def _scatter_nd_sc(data_hbm, idx_hbm, upd_hbm, out_hbm, *scratch):
    _body(data_hbm, idx_hbm, upd_hbm, out_hbm, *scratch)


def kernel(data, indices, updates):
    return _scatter_nd_sc(data, indices.reshape(B), updates)


# speedup vs baseline: 4.3508x; 4.3508x over previous
import functools

import jax
import jax.numpy as jnp
from jax import lax
from jax.experimental import pallas as pl
from jax.experimental.pallas import tpu as pltpu
from jax.experimental.pallas import tpu_sc as plsc

M = 1000000
D = 32
B = 16384

NC = 2
NS = 16
L = 16
NW = NC * NS
ROWS_MAIN = 31248
ROWS_MAX = ROWS_MAIN + 8
BLK = 1008
NBLK = ROWS_MAIN // BLK
NSEL_GROUPS = B // L

CAP = 1024
CHUNK = 128
WCAP = CAP + CHUNK


def _body(data_hbm, idx_hbm, upd_hbm, out_hbm,
          idx_v, blist, loclist, b_w, loc_w, claim, gbuf, brow, locrow,
          cbuf0, cbuf1,
          sin0, sin1, sout0, sout1, sgather, sscatter):
    wid = lax.axis_index("s") * NC + lax.axis_index("c")
    lo = wid * ROWS_MAIN + 8 * (wid // 4)
    hi = (wid + 1) * ROWS_MAIN + 8 * ((wid + 1) // 4)

    bufs = (cbuf0, cbuf1)
    sins = (sin0, sin1)
    souts = (sout0, sout1)

    cp_in = [
        pltpu.async_copy(data_hbm.at[pl.ds(lo, BLK)], cbuf0, sin0),
        pltpu.async_copy(data_hbm.at[pl.ds(lo + BLK, BLK)], cbuf1, sin1),
    ]

    pltpu.sync_copy(idx_hbm, idx_v)

    lane = lax.iota(jnp.int32, L)

    def sel_body(g, off):
        idxv = idx_v[pl.ds(g * L, L)]
        m = (idxv >= lo) & (idxv < hi)
        cnt = jnp.sum(m.astype(jnp.int32))
        safe = jnp.minimum(off, CAP)
        plsc.store_compressed(blist.at[pl.ds(safe, L)], g * L + lane, mask=m)
        plsc.store_compressed(loclist.at[pl.ds(safe, L)], idxv, mask=m)
        return off + cnt

    n_sel = lax.fori_loop(0, NSEL_GROUPS, sel_body, jnp.int32(0))
    n_sel = jnp.minimum(n_sel, CAP)

    def claim_body(g, _):
        jv = g * L + lane
        valid = jv < n_sel
        locv = loclist[pl.ds(g * L, L)]
        locl = jnp.where(valid, locv - lo, 0)
        for i in range(L):
            plsc.store_scatter(claim, [locl], jv, mask=valid & (lane == i))
        return 0

    lax.fori_loop(0, (n_sel + L - 1) // L, claim_body, 0)

    def win_body(g, offw):
        jv = g * L + lane
        valid = jv < n_sel
        locv = loclist[pl.ds(g * L, L)]
        bv = blist[pl.ds(g * L, L)]
        locl = jnp.where(valid, locv - lo, 0)
        cl = plsc.load_gather(claim, [locl], mask=valid)
        win = valid & (cl == jv)
        cnt = jnp.sum(win.astype(jnp.int32))
        plsc.store_compressed(b_w.at[pl.ds(offw, L)], bv, mask=win)
        plsc.store_compressed(loc_w.at[pl.ds(offw, L)], locv, mask=win)
        return offw + cnt

    n_w = lax.fori_loop(0, (n_sel + L - 1) // L, win_body, jnp.int32(0))

    nch = (n_w + CHUNK - 1) // CHUNK
    b0v = b_w[pl.ds(0, L)]
    l0v = loc_w[pl.ds(0, L)]
    b0 = jnp.full((L,), b0v[0], dtype=jnp.int32)
    l0 = jnp.full((L,), l0v[0], dtype=jnp.int32)
    for g in range(CHUNK // L):
        b_w[pl.ds(n_w + g * L, L)] = b0
        loc_w[pl.ds(n_w + g * L, L)] = l0

    cp_out = [None, None]
    for k in range(NBLK):
        p = k % 2
        cp_in[p].wait()
        cp_out[p] = pltpu.async_copy(
            bufs[p], out_hbm.at[pl.ds(lo + k * BLK, BLK)], souts[p])
        if k + 2 < NBLK:
            cp_out[p].wait()
            cp_in[p] = pltpu.async_copy(
                data_hbm.at[pl.ds(lo + (k + 2) * BLK, BLK)], bufs[p], sins[p])
    cp_out[(NBLK - 1) % 2].wait()
    cp_out[(NBLK - 2) % 2].wait()

    @pl.when(hi - lo > ROWS_MAIN)
    def _tail():
        tb = cbuf0.at[pl.ds(0, 8)]
        pltpu.sync_copy(data_hbm.at[pl.ds(lo + ROWS_MAIN, 8)], tb)
        pltpu.sync_copy(tb, out_hbm.at[pl.ds(lo + ROWS_MAIN, 8)])

    def apply_body(c, _):
        base = c * CHUNK
        for i in range(CHUNK // L):
            brow[pl.ds(i * L, L)] = b_w[pl.ds(base + i * L, L)]
            locrow[pl.ds(i * L, L)] = loc_w[pl.ds(base + i * L, L)]
        pltpu.async_copy(upd_hbm.at[brow], gbuf, sgather).wait()
        pltpu.async_copy(gbuf, out_hbm.at[locrow], sscatter).wait()
        return 0

    lax.fori_loop(0, nch, apply_body, 0)


@functools.partial(
    pl.kernel,
    out_type=jax.ShapeDtypeStruct((M, D), jnp.float32),
    mesh=plsc.VectorSubcoreMesh(
        core_axis_name="c", subcore_axis_name="s", num_cores=NC,
        num_subcores=NS),
    scratch_types=[
        pltpu.VMEM((B,), jnp.int32),
        pltpu.VMEM((CAP + L,), jnp.int32),
        pltpu.VMEM((CAP + L,), jnp.int32),
        pltpu.VMEM((WCAP,), jnp.int32),
        pltpu.VMEM((WCAP,), jnp.int32),
        pltpu.VMEM((ROWS_MAX,), jnp.int32),
        pltpu.VMEM((CHUNK, D), jnp.float32),
        pltpu.VMEM((CHUNK,), jnp.int32),
        pltpu.VMEM((CHUNK,), jnp.int32),
        pltpu.VMEM((BLK, D), jnp.float32),
        pltpu.VMEM((BLK, D), jnp.float32),
        pltpu.SemaphoreType.DMA,
        pltpu.SemaphoreType.DMA,
        pltpu.SemaphoreType.DMA,
        pltpu.SemaphoreType.DMA,
        pltpu.SemaphoreType.DMA,
        pltpu.SemaphoreType.DMA,
    ],
    compiler_params=pltpu.CompilerParams(
        needs_layout_passes=False, use_tc_tiling_on_sc=False),
)
def _scatter_nd_sc(data_hbm, idx_hbm, upd_hbm, out_hbm, *scratch):
    _body(data_hbm, idx_hbm, upd_hbm, out_hbm, *scratch)


def kernel(data, indices, updates):
    return _scatter_nd_sc(data, indices.reshape(B), updates)
